# baseline (speedup 1.0000x reference)
import functools

import jax
import jax.numpy as jnp
from jax import lax
from jax.experimental import pallas as pl
from jax.experimental.pallas import tpu as pltpu
from jax.experimental.pallas import tpu_sc as plsc

SUMMARY = 50
NC, NS = 2, 16
NW = NC * NS

VOCAB = 1000000
VOCAB_PAD = 1000064
D = 64
N_UNITS = VOCAB_PAD // 128
UNITS_PER_W = 123


def _make_transpose():
    mesh = plsc.VectorSubcoreMesh(core_axis_name="c", subcore_axis_name="s")

    @functools.partial(
        pl.kernel,
        out_type=jax.ShapeDtypeStruct((VOCAB_PAD // 2, 128), jnp.float32),
        mesh=mesh,
        scratch_types=[
            pltpu.VMEM((2, D, 128), jnp.float32),
            pltpu.VMEM((2, D, 128), jnp.float32),
            pltpu.SemaphoreType.DMA,
            pltpu.SemaphoreType.DMA,
            pltpu.SemaphoreType.DMA,
            pltpu.SemaphoreType.DMA,
        ],
        compiler_params=pltpu.CompilerParams(
            use_tc_tiling_on_sc=True, needs_layout_passes=False),
    )
    def transpose_kernel(tt_hbm, x_hbm, bin_v, bout_v, sr0, sr1, ss0, ss1):
        wid = lax.axis_index("s") * NC + lax.axis_index("c")
        sem_r, sem_s = (sr0, sr1), (ss0, ss1)
        iota = lax.iota(jnp.int32, 16)

        def start_read(b, u):
            pltpu.async_copy(
                tt_hbm.at[:, pl.ds(u * 128, 128)], bin_v.at[b], sem_r[b])

        def wait_read(b):
            pltpu.make_async_copy(
                tt_hbm.at[:, pl.ds(0, 128)], bin_v.at[b], sem_r[b]).wait()

        def start_store(b, u):
            pltpu.async_copy(
                bout_v.at[b], x_hbm.at[pl.ds(u * D, D)], sem_s[b])

        def wait_store(b):
            pltpu.make_async_copy(
                bout_v.at[b], x_hbm.at[pl.ds(0, D)], sem_s[b]).wait()

        for b in (0, 1):
            u0 = wid + 32 * b

            @pl.when(u0 < N_UNITS)
            def _():
                start_read(b, u0)

        def body(i, _):
            for b in (0, 1):
                u = wid + 32 * (2 * i + b)
                valid = u < N_UNITS

                @pl.when(valid)
                def _():
                    wait_read(b)

                    @pl.when(i > 0)
                    def _():
                        wait_store(b)

                    def row(r, carry):
                        for half in (0, 1):
                            v_spl = jnp.broadcast_to(2 * r + half, (16,))
                            for j0 in (0, 16, 32, 48):
                                g = plsc.load_gather(
                                    bin_v.at[b], [iota + j0, v_spl])
                                bout_v[b, r, pl.ds(half * 64 + j0, 16)] = g
                        return carry

                    lax.fori_loop(0, D, row, 0)
                    u_next = u + 2 * 32

                    @pl.when(u_next < N_UNITS)
                    def _():
                        start_read(b, u_next)

                    start_store(b, u)
            return 0

        lax.fori_loop(0, UNITS_PER_W, body, 0)
        for b in (0, 1):
            wait_store(b)

    return transpose_kernel


def _make_gather(total_rows, d, batch):
    rows_per_w = total_rows // NW
    chunk = 536
    n_chunks = rows_per_w // chunk
    assert chunk * n_chunks == rows_per_w and chunk % 8 == 0
    tb = batch // NW

    mesh = plsc.VectorSubcoreMesh(core_axis_name="c", subcore_axis_name="s")

    @functools.partial(
        pl.kernel,
        out_type=jax.ShapeDtypeStruct((total_rows, d), jnp.float32),
        mesh=mesh,
        scratch_types=[
            pltpu.VMEM((2, chunk), jnp.int32),
            pltpu.VMEM((2, chunk, d), jnp.float32),
            pltpu.VMEM((tb,), jnp.int32),
            pltpu.VMEM((tb, d), jnp.float32),
            pltpu.VMEM((d,), jnp.float32),
            pltpu.SemaphoreType.DMA,
            pltpu.SemaphoreType.DMA,
            pltpu.SemaphoreType.DMA,
            pltpu.SemaphoreType.DMA,
            pltpu.SemaphoreType.DMA,
            pltpu.SemaphoreType.DMA,
            pltpu.SemaphoreType.DMA,
        ],
        compiler_params=pltpu.CompilerParams(use_tc_tiling_on_sc=False),
    )
    def gather_kernel(idx_hbm, tidx_hbm, table_hbm, tv_hbm, out_hbm,
                      idx_v, rows_v, tidx_v, tbuf_v, tv_v,
                      si0, si1, sg0, sg1, ss0, ss1, st):
        wid = lax.axis_index("s") * NC + lax.axis_index("c")
        base = wid * rows_per_w
        sem_i, sem_g, sem_s = (si0, si1), (sg0, sg1), (ss0, ss1)

        def start_idx(c):
            off = base + c * chunk
            return pltpu.async_copy(
                idx_hbm.at[pl.ds(off, chunk)], idx_v.at[c % 2], sem_i[c % 2])

        def start_gather(c):
            return pltpu.async_copy(
                table_hbm.at[idx_v.at[c % 2]], rows_v.at[c % 2], sem_g[c % 2])

        def start_store(c):
            off = base + c * chunk
            return pltpu.async_copy(
                rows_v.at[c % 2], out_hbm.at[pl.ds(off, chunk)], sem_s[c % 2])

        idx_d = {0: start_idx(0), 1: start_idx(1)}
        tidx_d = pltpu.async_copy(
            tidx_hbm.at[pl.ds(wid * tb, tb)], tidx_v, st)
        pltpu.sync_copy(tv_hbm, tv_v)
        for k in range(d // 16):
            seg = tv_v[pl.ds(k * 16, 16)]
            for b in range(tb):
                tbuf_v[b, pl.ds(k * 16, 16)] = seg
        tidx_d.wait()

        gat_d, sto_d = {}, {}
        for c in range(n_chunks):
            idx_d[c].wait()
            if c >= 2:
                sto_d[c - 2].wait()
            gat_d[c] = start_gather(c)
            gat_d[c].wait()
            sto_d[c] = start_store(c)
            if c + 2 < n_chunks:
                idx_d[c + 2] = start_idx(c + 2)
        sto_d[n_chunks - 2].wait()
        sto_d[n_chunks - 1].wait()

        pltpu.async_copy(tbuf_v, out_hbm.at[tidx_v], st).wait()

    return gather_kernel


def kernel(x, table, type_embedding):
    b, seq = x.shape
    vocab, d = table.shape
    s = SUMMARY
    idx_full = jnp.concatenate(
        [x[:, : seq - s],
         jnp.zeros((b, 1), jnp.int32),
         x[:, seq - s:]], axis=1).reshape(-1)
    tidx = jnp.arange(b, dtype=jnp.int32) * (seq + 1) + (seq - s)
    tv = type_embedding.reshape(d)

    transpose_kernel = _make_transpose()
    xt = transpose_kernel(table.T)
    xr = xt.reshape(VOCAB_PAD, d)

    gather_kernel = _make_gather(b * (seq + 1), d, b)
    out_flat = gather_kernel(idx_full, tidx, xr, tv)
    return out_flat.reshape(b, seq + 1, d)


# speedup vs baseline: 1.3919x; 1.3919x over previous
import functools

import jax
import jax.numpy as jnp
from jax import lax
from jax.experimental import pallas as pl
from jax.experimental.pallas import tpu as pltpu
from jax.experimental.pallas import tpu_sc as plsc

SUMMARY = 50
NC, NS = 2, 16
NW = NC * NS

CHUNK = 256


def _make_gather(total_rows, d, batch, seq1):
    rows_per_w = total_rows // NW
    tb = batch // NW
    assert rows_per_w == tb * seq1
    n_chunks = -(-rows_per_w // CHUNK)
    tail_valid = rows_per_w - (n_chunks - 1) * CHUNK
    patches = [[] for _ in range(n_chunks)]
    for j in range(tb):
        lpos = j * seq1 + (seq1 - 1 - SUMMARY)
        patches[lpos // CHUNK].append(lpos % CHUNK)

    mesh = plsc.VectorSubcoreMesh(core_axis_name="c", subcore_axis_name="s")

    @functools.partial(
        pl.kernel,
        out_type=jax.ShapeDtypeStruct((total_rows // 2, 128), jnp.float32),
        mesh=mesh,
        scratch_types=[
            pltpu.VMEM((CHUNK,), jnp.int32),
            pltpu.VMEM((CHUNK,), jnp.int32),
            pltpu.VMEM((CHUNK,), jnp.int32),
            pltpu.VMEM((CHUNK,), jnp.int32),
            pltpu.VMEM((CHUNK,), jnp.int32),
            pltpu.VMEM((CHUNK,), jnp.int32),
            pltpu.VMEM((CHUNK, 128), jnp.float32),
            pltpu.VMEM((CHUNK, 128), jnp.float32),
            pltpu.VMEM((CHUNK // 2, 128), jnp.float32),
            pltpu.VMEM((CHUNK // 2, 128), jnp.float32),
            pltpu.VMEM((128,), jnp.float32),
            pltpu.SemaphoreType.DMA,
            pltpu.SemaphoreType.DMA,
            pltpu.SemaphoreType.DMA,
            pltpu.SemaphoreType.DMA,
            pltpu.SemaphoreType.DMA,
            pltpu.SemaphoreType.DMA,
            pltpu.SemaphoreType.DMA,
        ],
        compiler_params=pltpu.CompilerParams(
            use_tc_tiling_on_sc=True, needs_layout_passes=False),
    )
    def gather_kernel(idx_hbm, xp_hbm, tv_hbm, out_hbm,
                      ix0, ix1, hf0, hf1, pr0, pr1, rw0, rw1, cb0, cb1, tv_v,
                      si0, si1, sg0, sg1, ss0, ss1, st):
        idx_v, half_v, par_v = (ix0, ix1), (hf0, hf1), (pr0, pr1)
        rows_v, cbuf_v = (rw0, rw1), (cb0, cb1)
        wid = lax.axis_index("s") * NC + lax.axis_index("c")
        base = wid * rows_per_w
        obase = wid * (rows_per_w // 2)
        sem_i, sem_g, sem_s = (si0, si1), (sg0, sg1), (ss0, ss1)
        iota = lax.iota(jnp.int32, 16)
        half_iota = iota >> 1
        p64_iota = (iota & 1) * 64

        def start_idx(c):
            pltpu.async_copy(idx_hbm.at[pl.ds(base + c * CHUNK, CHUNK)],
                             idx_v[c % 2], sem_i[c % 2])

        def wait_idx(c):
            pltpu.make_async_copy(idx_hbm.at[pl.ds(0, CHUNK)],
                                  idx_v[c % 2], sem_i[c % 2]).wait()

        def prep_idx(c):
            s = c % 2

            def grp(g, carry):
                v = idx_v[s][pl.ds(g * 16, 16)]
                half_v[s][pl.ds(g * 16, 16)] = v >> 1
                par_v[s][pl.ds(g * 16, 16)] = (v & 1) * 64
                return carry

            lax.fori_loop(0, CHUNK // 16, grp, 0)

        def start_gather(c):
            pltpu.async_copy(xp_hbm.at[half_v[c % 2]],
                             rows_v[c % 2], sem_g[c % 2])

        def wait_gather(c):
            pltpu.make_async_copy(xp_hbm.at[pl.ds(0, CHUNK)],
                                  rows_v[c % 2], sem_g[c % 2]).wait()

        def compact(c):
            s = c % 2

            def grp(g, carry):
                par = par_v[s][pl.ds(g * 16, 16)]
                tvec = iota + g * 16
                rvec = half_iota + g * 8

                def feat(c0, carry2):
                    v = plsc.load_gather(rows_v[s], [tvec, par + c0])
                    plsc.store_scatter(cbuf_v[s], [rvec, p64_iota + c0], v)
                    return carry2

                lax.fori_loop(0, d, feat, 0)
                return carry

            lax.fori_loop(0, CHUNK // 16, grp, 0)
            for q in patches[c]:
                for k in range(d // 16):
                    cbuf_v[s][q // 2, pl.ds((q % 2) * 64 + k * 16, 16)] = (
                        tv_v[pl.ds(k * 16, 16)])

        def start_store(c):
            sz = (CHUNK if c < n_chunks - 1 else tail_valid) // 2
            pltpu.async_copy(cbuf_v[c % 2].at[pl.ds(0, sz)],
                             out_hbm.at[pl.ds(obase + c * CHUNK // 2, sz)],
                             sem_s[c % 2])

        def wait_store(c):
            sz = (CHUNK if c < n_chunks - 1 else tail_valid) // 2
            pltpu.make_async_copy(cbuf_v[c % 2].at[pl.ds(0, sz)],
                                  out_hbm.at[pl.ds(0, sz)],
                                  sem_s[c % 2]).wait()

        start_idx(0)
        start_idx(1)
        pltpu.async_copy(tv_hbm, tv_v, st).wait()
        wait_idx(0)
        prep_idx(0)
        start_gather(0)

        for c in range(n_chunks):
            if c + 1 < n_chunks:
                wait_idx(c + 1)
                prep_idx(c + 1)
                start_gather(c + 1)
            if c + 2 < n_chunks:
                start_idx(c + 2)
            wait_gather(c)
            if c >= 2:
                wait_store(c - 2)
            compact(c)
            start_store(c)
        wait_store(n_chunks - 2)
        wait_store(n_chunks - 1)

    return gather_kernel


def kernel(x, table, type_embedding):
    b, seq = x.shape
    vocab, d = table.shape
    s = SUMMARY
    idx_full = jnp.concatenate(
        [x[:, : seq - s],
         jnp.zeros((b, 1), jnp.int32),
         x[:, seq - s:]], axis=1).reshape(-1)
    idx_pad = jnp.concatenate([idx_full, jnp.zeros((CHUNK,), jnp.int32)])
    tv = jnp.concatenate(
        [type_embedding.reshape(d), jnp.zeros((128 - d,), jnp.float32)])
    xp = table.reshape(vocab // 2, 2 * d)

    gather_kernel = _make_gather(b * (seq + 1), d, b, seq + 1)
    out128 = gather_kernel(idx_pad, xp, tv)
    return out128.reshape(b, seq + 1, d)


# speedup vs baseline: 1.6991x; 1.2208x over previous
import functools

import jax
import jax.numpy as jnp
from jax import lax
from jax.experimental import pallas as pl
from jax.experimental.pallas import tpu as pltpu
from jax.experimental.pallas import tpu_sc as plsc

SUMMARY = 50
NC, NS = 2, 16
NW = NC * NS

VOCAB = 1000000
VOCAB_PAD = 1000064
D = 64
N_UNITS = VOCAB_PAD // 128
UNITS_PER_W = 123


def _make_transpose():
    mesh = plsc.VectorSubcoreMesh(core_axis_name="c", subcore_axis_name="s")

    @functools.partial(
        pl.kernel,
        out_type=jax.ShapeDtypeStruct((VOCAB_PAD // 2, 128), jnp.float32),
        mesh=mesh,
        scratch_types=[
            pltpu.VMEM((2, D, 128), jnp.float32),
            pltpu.VMEM((2, D, 128), jnp.float32),
            pltpu.SemaphoreType.DMA,
            pltpu.SemaphoreType.DMA,
            pltpu.SemaphoreType.DMA,
            pltpu.SemaphoreType.DMA,
        ],
        compiler_params=pltpu.CompilerParams(
            use_tc_tiling_on_sc=True, needs_layout_passes=False),
    )
    def transpose_kernel(tt_hbm, x_hbm, bin_v, bout_v, sr0, sr1, ss0, ss1):
        wid = lax.axis_index("s") * NC + lax.axis_index("c")
        sem_r, sem_s = (sr0, sr1), (ss0, ss1)
        iota = lax.iota(jnp.int32, 16)
        c_vecs = [iota + j0 for j0 in (0, 16, 32, 48)]

        def start_read(b, u):
            pltpu.async_copy(
                tt_hbm.at[:, pl.ds(u * 128, 128)], bin_v.at[b], sem_r[b])

        def wait_read(b):
            pltpu.make_async_copy(
                tt_hbm.at[:, pl.ds(0, 128)], bin_v.at[b], sem_r[b]).wait()

        def start_store(b, u):
            pltpu.async_copy(
                bout_v.at[b], x_hbm.at[pl.ds(u * D, D)], sem_s[b])

        def wait_store(b):
            pltpu.make_async_copy(
                bout_v.at[b], x_hbm.at[pl.ds(0, D)], sem_s[b]).wait()

        for b in (0, 1):
            u0 = wid + 32 * b

            @pl.when(u0 < N_UNITS)
            def _():
                start_read(b, u0)

        def body(i, _):
            for b in (0, 1):
                u = wid + 32 * (2 * i + b)
                valid = u < N_UNITS

                @pl.when(valid)
                def _():
                    wait_read(b)

                    @pl.when(i > 0)
                    def _():
                        wait_store(b)

                    @plsc.parallel_loop(0, D, unroll=8)
                    def row(r):
                        for half in (0, 1):
                            v_spl = jnp.broadcast_to(2 * r + half, (16,))
                            for jb in range(4):
                                g = plsc.load_gather(
                                    bin_v.at[b], [c_vecs[jb], v_spl])
                                bout_v[b, r, pl.ds(half * 64 + jb * 16, 16)] = g
                    u_next = u + 2 * 32

                    @pl.when(u_next < N_UNITS)
                    def _():
                        start_read(b, u_next)

                    start_store(b, u)
            return 0

        lax.fori_loop(0, UNITS_PER_W, body, 0)
        for b in (0, 1):
            wait_store(b)

    return transpose_kernel


def _make_gather(total_rows, d, batch):
    rows_per_w = total_rows // NW
    chunk = 536
    n_chunks = rows_per_w // chunk
    assert chunk * n_chunks == rows_per_w and chunk % 8 == 0
    tb = batch // NW

    mesh = plsc.VectorSubcoreMesh(core_axis_name="c", subcore_axis_name="s")

    @functools.partial(
        pl.kernel,
        out_type=jax.ShapeDtypeStruct((total_rows, d), jnp.float32),
        mesh=mesh,
        scratch_types=[
            pltpu.VMEM((2, chunk), jnp.int32),
            pltpu.VMEM((2, chunk, d), jnp.float32),
            pltpu.VMEM((tb,), jnp.int32),
            pltpu.VMEM((tb, d), jnp.float32),
            pltpu.VMEM((d,), jnp.float32),
            pltpu.SemaphoreType.DMA,
            pltpu.SemaphoreType.DMA,
            pltpu.SemaphoreType.DMA,
            pltpu.SemaphoreType.DMA,
            pltpu.SemaphoreType.DMA,
            pltpu.SemaphoreType.DMA,
            pltpu.SemaphoreType.DMA,
        ],
        compiler_params=pltpu.CompilerParams(use_tc_tiling_on_sc=False),
    )
    def gather_kernel(idx_hbm, tidx_hbm, table_hbm, tv_hbm, out_hbm,
                      idx_v, rows_v, tidx_v, tbuf_v, tv_v,
                      si0, si1, sg0, sg1, ss0, ss1, st):
        wid = lax.axis_index("s") * NC + lax.axis_index("c")
        base = wid * rows_per_w
        sem_i, sem_g, sem_s = (si0, si1), (sg0, sg1), (ss0, ss1)

        def start_idx(c):
            off = base + c * chunk
            return pltpu.async_copy(
                idx_hbm.at[pl.ds(off, chunk)], idx_v.at[c % 2], sem_i[c % 2])

        def start_gather(c):
            return pltpu.async_copy(
                table_hbm.at[idx_v.at[c % 2]], rows_v.at[c % 2], sem_g[c % 2])

        def start_store(c):
            off = base + c * chunk
            return pltpu.async_copy(
                rows_v.at[c % 2], out_hbm.at[pl.ds(off, chunk)], sem_s[c % 2])

        idx_d = {0: start_idx(0), 1: start_idx(1)}
        tidx_d = pltpu.async_copy(
            tidx_hbm.at[pl.ds(wid * tb, tb)], tidx_v, st)
        pltpu.sync_copy(tv_hbm, tv_v)
        for k in range(d // 16):
            seg = tv_v[pl.ds(k * 16, 16)]
            for b in range(tb):
                tbuf_v[b, pl.ds(k * 16, 16)] = seg
        tidx_d.wait()

        gat_d, sto_d = {}, {}
        for c in range(n_chunks):
            idx_d[c].wait()
            if c >= 2:
                sto_d[c - 2].wait()
            gat_d[c] = start_gather(c)
            gat_d[c].wait()
            sto_d[c] = start_store(c)
            if c + 2 < n_chunks:
                idx_d[c + 2] = start_idx(c + 2)
        sto_d[n_chunks - 2].wait()
        sto_d[n_chunks - 1].wait()

        pltpu.async_copy(tbuf_v, out_hbm.at[tidx_v], st).wait()

    return gather_kernel


def kernel(x, table, type_embedding):
    b, seq = x.shape
    vocab, d = table.shape
    s = SUMMARY
    idx_full = jnp.concatenate(
        [x[:, : seq - s],
         jnp.zeros((b, 1), jnp.int32),
         x[:, seq - s:]], axis=1).reshape(-1)
    tidx = jnp.arange(b, dtype=jnp.int32) * (seq + 1) + (seq - s)
    tv = type_embedding.reshape(d)

    transpose_kernel = _make_transpose()
    xt = transpose_kernel(table.T)
    xr = xt.reshape(VOCAB_PAD, d)

    gather_kernel = _make_gather(b * (seq + 1), d, b)
    out_flat = gather_kernel(idx_full, tidx, xr, tv)
    return out_flat.reshape(b, seq + 1, d)


# speedup vs baseline: 3.4959x; 2.0575x over previous
import functools

import jax
import jax.numpy as jnp
from jax import lax
from jax.experimental import pallas as pl
from jax.experimental.pallas import tpu as pltpu
from jax.experimental.pallas import tpu_sc as plsc

SUMMARY = 50
NC, NS = 2, 16
NW = NC * NS

VOCAB = 1000000
VOCAB_PAD = 1000064
D = 64
N_UNITS = VOCAB_PAD // 128
UNITS_PER_W = 123


def _make_transpose():
    mesh = plsc.VectorSubcoreMesh(core_axis_name="c", subcore_axis_name="s")

    @functools.partial(
        pl.kernel,
        out_type=jax.ShapeDtypeStruct((VOCAB_PAD * D,), jnp.float32),
        mesh=mesh,
        scratch_types=[
            pltpu.VMEM((D, 128), jnp.float32),
            pltpu.VMEM((D, 128), jnp.float32),
            pltpu.VMEM((D * 128,), jnp.float32),
            pltpu.VMEM((D * 128,), jnp.float32),
            pltpu.SemaphoreType.DMA,
            pltpu.SemaphoreType.DMA,
            pltpu.SemaphoreType.DMA,
            pltpu.SemaphoreType.DMA,
        ],
        compiler_params=pltpu.CompilerParams(
            use_tc_tiling_on_sc=True, needs_layout_passes=False),
    )
    def transpose_kernel(tt_hbm, x_hbm, bi0, bi1, bo0, bo1,
                         sr0, sr1, ss0, ss1):
        bin_v, bout_v = (bi0, bi1), (bo0, bo1)
        wid = lax.axis_index("s") * NC + lax.axis_index("c")
        sem_r, sem_s = (sr0, sr1), (ss0, ss1)
        iota = lax.iota(jnp.int32, 16)
        m_vecs = [(iota + k) & 15 for k in range(16)]
        d_vecs = [(m >> 1) * 128 + (m & 1) * 64 + iota for m in m_vecs]

        def start_read(b, u):
            pltpu.async_copy(
                tt_hbm.at[:, pl.ds(u * 128, 128)], bin_v[b], sem_r[b])

        def wait_read(b):
            pltpu.make_async_copy(
                tt_hbm.at[:, pl.ds(0, 128)], bin_v[b], sem_r[b]).wait()

        def start_store(b, u):
            pltpu.async_copy(
                bout_v[b], x_hbm.at[pl.ds(u * D * 128, D * 128)], sem_s[b])

        def wait_store(b):
            pltpu.make_async_copy(
                bout_v[b], x_hbm.at[pl.ds(0, D * 128)], sem_s[b]).wait()

        for b in (0, 1):
            u0 = wid + 32 * b

            @pl.when(u0 < N_UNITS)
            def _():
                start_read(b, u0)

        def body(i, _):
            for b in (0, 1):
                u = wid + 32 * (2 * i + b)
                valid = u < N_UNITS

                @pl.when(valid)
                def _():
                    wait_read(b)

                    @pl.when(i > 0)
                    def _():
                        wait_store(b)

                    @plsc.parallel_loop(0, 32, unroll=2)
                    def blk(t):
                        vb = t >> 2
                        cb = t & 3
                        cvec = iota + cb * 16
                        v0 = vb * 16
                        dbase = vb * 1024 + cb * 16
                        src_ref = bin_v[b]
                        dst_ref = bout_v[b]
                        for k in range(16):
                            g = plsc.load_gather(
                                src_ref, [cvec, m_vecs[k] + v0])
                            plsc.store_scatter(
                                dst_ref, [d_vecs[k] + dbase], g)
                    u_next = u + 2 * 32

                    @pl.when(u_next < N_UNITS)
                    def _():
                        start_read(b, u_next)

                    start_store(b, u)
            return 0

        lax.fori_loop(0, UNITS_PER_W, body, 0)
        for b in (0, 1):
            wait_store(b)

    return transpose_kernel


def _make_gather(total_rows, d, batch):
    rows_per_w = total_rows // NW
    chunk = 536
    n_chunks = rows_per_w // chunk
    assert chunk * n_chunks == rows_per_w and chunk % 8 == 0
    tb = batch // NW

    mesh = plsc.VectorSubcoreMesh(core_axis_name="c", subcore_axis_name="s")

    @functools.partial(
        pl.kernel,
        out_type=jax.ShapeDtypeStruct((total_rows, d), jnp.float32),
        mesh=mesh,
        scratch_types=[
            pltpu.VMEM((2, chunk), jnp.int32),
            pltpu.VMEM((2, chunk, d), jnp.float32),
            pltpu.VMEM((tb,), jnp.int32),
            pltpu.VMEM((tb, d), jnp.float32),
            pltpu.VMEM((d,), jnp.float32),
            pltpu.SemaphoreType.DMA,
            pltpu.SemaphoreType.DMA,
            pltpu.SemaphoreType.DMA,
            pltpu.SemaphoreType.DMA,
            pltpu.SemaphoreType.DMA,
            pltpu.SemaphoreType.DMA,
            pltpu.SemaphoreType.DMA,
        ],
        compiler_params=pltpu.CompilerParams(use_tc_tiling_on_sc=False),
    )
    def gather_kernel(idx_hbm, tidx_hbm, table_hbm, tv_hbm, out_hbm,
                      idx_v, rows_v, tidx_v, tbuf_v, tv_v,
                      si0, si1, sg0, sg1, ss0, ss1, st):
        wid = lax.axis_index("s") * NC + lax.axis_index("c")
        base = wid * rows_per_w
        sem_i, sem_g, sem_s = (si0, si1), (sg0, sg1), (ss0, ss1)

        def start_idx(c):
            off = base + c * chunk
            return pltpu.async_copy(
                idx_hbm.at[pl.ds(off, chunk)], idx_v.at[c % 2], sem_i[c % 2])

        def start_gather(c):
            return pltpu.async_copy(
                table_hbm.at[idx_v.at[c % 2]], rows_v.at[c % 2], sem_g[c % 2])

        def start_store(c):
            off = base + c * chunk
            return pltpu.async_copy(
                rows_v.at[c % 2], out_hbm.at[pl.ds(off, chunk)], sem_s[c % 2])

        idx_d = {0: start_idx(0), 1: start_idx(1)}
        tidx_d = pltpu.async_copy(
            tidx_hbm.at[pl.ds(wid * tb, tb)], tidx_v, st)
        pltpu.sync_copy(tv_hbm, tv_v)
        for k in range(d // 16):
            seg = tv_v[pl.ds(k * 16, 16)]
            for b in range(tb):
                tbuf_v[b, pl.ds(k * 16, 16)] = seg
        tidx_d.wait()

        gat_d, sto_d = {}, {}
        for c in range(n_chunks):
            idx_d[c].wait()
            if c >= 2:
                sto_d[c - 2].wait()
            gat_d[c] = start_gather(c)
            gat_d[c].wait()
            sto_d[c] = start_store(c)
            if c + 2 < n_chunks:
                idx_d[c + 2] = start_idx(c + 2)
        sto_d[n_chunks - 2].wait()
        sto_d[n_chunks - 1].wait()

        pltpu.async_copy(tbuf_v, out_hbm.at[tidx_v], st).wait()

    return gather_kernel


def kernel(x, table, type_embedding):
    b, seq = x.shape
    vocab, d = table.shape
    s = SUMMARY
    idx_full = jnp.concatenate(
        [x[:, : seq - s],
         jnp.zeros((b, 1), jnp.int32),
         x[:, seq - s:]], axis=1).reshape(-1)
    tidx = jnp.arange(b, dtype=jnp.int32) * (seq + 1) + (seq - s)
    tv = type_embedding.reshape(d)

    transpose_kernel = _make_transpose()
    xt = transpose_kernel(table.T)
    xr = xt.reshape(VOCAB_PAD, d)

    gather_kernel = _make_gather(b * (seq + 1), d, b)
    out_flat = gather_kernel(idx_full, tidx, xr, tv)
    return out_flat.reshape(b, seq + 1, d)


# speedup vs baseline: 3.8116x; 1.0903x over previous
import functools

import jax
import jax.numpy as jnp
from jax import lax
from jax.experimental import pallas as pl
from jax.experimental.pallas import tpu as pltpu
from jax.experimental.pallas import tpu_sc as plsc

SUMMARY = 50
NC, NS = 2, 16
NW = NC * NS

VOCAB = 1000000
VOCAB_PAD = 1000064
D = 64
N_UNITS = VOCAB_PAD // 128
UNITS_PER_W = 123


def _make_transpose():
    mesh = plsc.VectorSubcoreMesh(core_axis_name="c", subcore_axis_name="s")

    @functools.partial(
        pl.kernel,
        out_type=jax.ShapeDtypeStruct((VOCAB_PAD * D,), jnp.float32),
        mesh=mesh,
        scratch_types=[
            pltpu.VMEM((D, 128), jnp.float32),
            pltpu.VMEM((D, 128), jnp.float32),
            pltpu.VMEM((D * 128,), jnp.float32),
            pltpu.VMEM((D * 128,), jnp.float32),
            pltpu.SemaphoreType.DMA,
            pltpu.SemaphoreType.DMA,
            pltpu.SemaphoreType.DMA,
            pltpu.SemaphoreType.DMA,
        ],
        compiler_params=pltpu.CompilerParams(
            use_tc_tiling_on_sc=True, needs_layout_passes=False),
    )
    def transpose_kernel(tt_hbm, x_hbm, bi0, bi1, bo0, bo1,
                         sr0, sr1, ss0, ss1):
        bin_v, bout_v = (bi0, bi1), (bo0, bo1)
        wid = lax.axis_index("s") * NC + lax.axis_index("c")
        sem_r, sem_s = (sr0, sr1), (ss0, ss1)
        iota = lax.iota(jnp.int32, 16)
        m_vecs = [(iota + k) & 15 for k in range(16)]
        d_vecs = [(m >> 1) * 128 + (m & 1) * 64 + iota for m in m_vecs]

        def start_read(b, u):
            pltpu.async_copy(
                tt_hbm.at[:, pl.ds(u * 128, 128)], bin_v[b], sem_r[b])

        def wait_read(b):
            pltpu.make_async_copy(
                tt_hbm.at[:, pl.ds(0, 128)], bin_v[b], sem_r[b]).wait()

        def start_store(b, u):
            pltpu.async_copy(
                bout_v[b], x_hbm.at[pl.ds(u * D * 128, D * 128)], sem_s[b])

        def wait_store(b):
            pltpu.make_async_copy(
                bout_v[b], x_hbm.at[pl.ds(0, D * 128)], sem_s[b]).wait()

        for b in (0, 1):
            u0 = wid + 32 * b

            @pl.when(u0 < N_UNITS)
            def _():
                start_read(b, u0)

        def body(i, _):
            for b in (0, 1):
                u = wid + 32 * (2 * i + b)
                valid = u < N_UNITS

                @pl.when(valid)
                def _():
                    wait_read(b)

                    @pl.when(i > 0)
                    def _():
                        wait_store(b)

                    @plsc.parallel_loop(0, 32, unroll=4)
                    def blk(t):
                        vb = t >> 2
                        cb = t & 3
                        cvec = iota + cb * 16
                        v0 = vb * 16
                        dbase = vb * 1024 + cb * 16
                        src_ref = bin_v[b]
                        dst_ref = bout_v[b]
                        for k in range(16):
                            g = plsc.load_gather(
                                src_ref, [cvec, m_vecs[k] + v0])
                            plsc.store_scatter(
                                dst_ref, [d_vecs[k] + dbase], g)
                    u_next = u + 2 * 32

                    @pl.when(u_next < N_UNITS)
                    def _():
                        start_read(b, u_next)

                    start_store(b, u)
            return 0

        lax.fori_loop(0, UNITS_PER_W, body, 0)
        for b in (0, 1):
            wait_store(b)

    return transpose_kernel


def _make_gather(total_rows, d, batch):
    rows_per_w = total_rows // NW
    chunk = 536
    n_chunks = rows_per_w // chunk
    assert chunk * n_chunks == rows_per_w and chunk % 8 == 0
    tb = batch // NW

    mesh = plsc.VectorSubcoreMesh(core_axis_name="c", subcore_axis_name="s")

    @functools.partial(
        pl.kernel,
        out_type=jax.ShapeDtypeStruct((total_rows, d), jnp.float32),
        mesh=mesh,
        scratch_types=[
            pltpu.VMEM((2, chunk), jnp.int32),
            pltpu.VMEM((2, chunk, d), jnp.float32),
            pltpu.VMEM((tb,), jnp.int32),
            pltpu.VMEM((tb, d), jnp.float32),
            pltpu.VMEM((d,), jnp.float32),
            pltpu.SemaphoreType.DMA,
            pltpu.SemaphoreType.DMA,
            pltpu.SemaphoreType.DMA,
            pltpu.SemaphoreType.DMA,
            pltpu.SemaphoreType.DMA,
            pltpu.SemaphoreType.DMA,
            pltpu.SemaphoreType.DMA,
        ],
        compiler_params=pltpu.CompilerParams(use_tc_tiling_on_sc=False),
    )
    def gather_kernel(idx_hbm, tidx_hbm, table_hbm, tv_hbm, out_hbm,
                      idx_v, rows_v, tidx_v, tbuf_v, tv_v,
                      si0, si1, sg0, sg1, ss0, ss1, st):
        wid = lax.axis_index("s") * NC + lax.axis_index("c")
        base = wid * rows_per_w
        sem_i, sem_g, sem_s = (si0, si1), (sg0, sg1), (ss0, ss1)

        def start_idx(c):
            off = base + c * chunk
            return pltpu.async_copy(
                idx_hbm.at[pl.ds(off, chunk)], idx_v.at[c % 2], sem_i[c % 2])

        def start_gather(c):
            return pltpu.async_copy(
                table_hbm.at[idx_v.at[c % 2]], rows_v.at[c % 2], sem_g[c % 2])

        def start_store(c):
            off = base + c * chunk
            return pltpu.async_copy(
                rows_v.at[c % 2], out_hbm.at[pl.ds(off, chunk)], sem_s[c % 2])

        idx_d = {0: start_idx(0), 1: start_idx(1)}
        tidx_d = pltpu.async_copy(
            tidx_hbm.at[pl.ds(wid * tb, tb)], tidx_v, st)
        pltpu.sync_copy(tv_hbm, tv_v)
        for k in range(d // 16):
            seg = tv_v[pl.ds(k * 16, 16)]
            for b in range(tb):
                tbuf_v[b, pl.ds(k * 16, 16)] = seg
        tidx_d.wait()

        gat_d, sto_d = {}, {}
        for c in range(n_chunks):
            idx_d[c].wait()
            if c >= 2:
                sto_d[c - 2].wait()
            gat_d[c] = start_gather(c)
            gat_d[c].wait()
            sto_d[c] = start_store(c)
            if c + 2 < n_chunks:
                idx_d[c + 2] = start_idx(c + 2)
        sto_d[n_chunks - 2].wait()
        sto_d[n_chunks - 1].wait()

        pltpu.async_copy(tbuf_v, out_hbm.at[tidx_v], st).wait()

    return gather_kernel


def kernel(x, table, type_embedding):
    b, seq = x.shape
    vocab, d = table.shape
    s = SUMMARY
    idx_full = jnp.concatenate(
        [x[:, : seq - s],
         jnp.zeros((b, 1), jnp.int32),
         x[:, seq - s:]], axis=1).reshape(-1)
    tidx = jnp.arange(b, dtype=jnp.int32) * (seq + 1) + (seq - s)
    tv = type_embedding.reshape(d)

    transpose_kernel = _make_transpose()
    xt = transpose_kernel(table.T)
    xr = xt.reshape(VOCAB_PAD, d)

    gather_kernel = _make_gather(b * (seq + 1), d, b)
    out_flat = gather_kernel(idx_full, tidx, xr, tv)
    return out_flat.reshape(b, seq + 1, d)
